# baseline (speedup 1.0000x reference)
import functools

import jax
import jax.numpy as jnp
from jax.experimental import pallas as pl


@functools.lru_cache(maxsize=None)
def _eps_const_t(n, e):
    return jax.random.normal(jax.random.key(42), (n, e), dtype=jnp.float32).T


def _route_block(xb, wc, bc, eps):
    logits = jax.lax.dot_general(
        wc, xb, (((1,), (1,)), ((), ())),
        preferred_element_type=jnp.float32) + bc
    e = eps.shape[0]
    r = logits[:e]
    nl = logits[e:]
    noisy = r + eps * jnp.logaddexp(nl, 0.0)

    sub = jax.lax.broadcasted_iota(jnp.int32, noisy.shape, 0)
    m0 = jnp.max(noisy, axis=0, keepdims=True)
    i0 = jnp.min(jnp.where(noisy == m0, sub, e), axis=0, keepdims=True)
    masked = jnp.where(sub == i0, -jnp.inf, noisy)
    m1 = jnp.max(masked, axis=0, keepdims=True)
    i1 = jnp.min(jnp.where(masked == m1, sub, e), axis=0, keepdims=True)

    d = jnp.exp(m1 - m0)
    p0 = 1.0 / (1.0 + d)
    p1 = d / (1.0 + d)
    rout_t = jnp.where(sub == i0, p0, 0.0) + jnp.where(sub == i1, p1, 0.0)
    idx_t = jnp.concatenate([i0, i1], axis=0)
    return rout_t.T, idx_t.T


def _router_kernel(xa_ref, xb_ref, wc_ref, bc_ref, epsa_ref, epsb_ref,
                   rout_a_ref, idx_a_ref, rout_b_ref, idx_b_ref):
    wc = wc_ref[...]
    bc = bc_ref[...]
    rout_a_ref[...], idx_a_ref[...] = _route_block(
        xa_ref[...], wc, bc, epsa_ref[...])
    rout_b_ref[...], idx_b_ref[...] = _route_block(
        xb_ref[...], wc, bc, epsb_ref[...])


def kernel(x, W_route, b_route, W_noise, b_noise):
    n, dim = x.shape
    e = W_route.shape[0]
    eps_t = _eps_const_t(n, e)
    wc = jnp.concatenate([W_route, W_noise], axis=0)
    bc = jnp.concatenate([b_route, b_noise]).reshape(2 * e, 1)
    blk = 2048
    nb2 = (n // 2) // blk
    outs = pl.pallas_call(
        _router_kernel,
        grid=(nb2,),
        in_specs=[
            pl.BlockSpec((blk, dim), lambda i: (i, 0)),
            pl.BlockSpec((blk, dim), lambda i: (i + nb2, 0)),
            pl.BlockSpec((2 * e, dim), lambda i: (0, 0)),
            pl.BlockSpec((2 * e, 1), lambda i: (0, 0)),
            pl.BlockSpec((e, blk), lambda i: (0, i)),
            pl.BlockSpec((e, blk), lambda i: (0, i + nb2)),
        ],
        out_specs=(
            pl.BlockSpec((blk, e), lambda i: (i, 0)),
            pl.BlockSpec((blk, 2), lambda i: (i, 0)),
            pl.BlockSpec((blk, e), lambda i: (i, 0)),
            pl.BlockSpec((blk, 2), lambda i: (i, 0)),
        ),
        out_shape=(
            jax.ShapeDtypeStruct((n // 2, e), jnp.float32),
            jax.ShapeDtypeStruct((n // 2, 2), jnp.int32),
            jax.ShapeDtypeStruct((n // 2, e), jnp.float32),
            jax.ShapeDtypeStruct((n // 2, 2), jnp.int32),
        ),
    )(x, x, wc, bc, eps_t, eps_t)
    rout = jnp.concatenate([outs[0], outs[2]], axis=0)
    idx = jnp.concatenate([outs[1], outs[3]], axis=0)
    return (rout, idx)


# speedup vs baseline: 1.0422x; 1.0422x over previous
import functools

import jax
import jax.numpy as jnp
from jax.experimental import pallas as pl


@functools.lru_cache(maxsize=None)
def _eps_const_t(n, e):
    return jax.random.normal(jax.random.key(42), (n, e), dtype=jnp.float32).T


def _router_kernel(x_ref, wc_ref, bc_ref, eps_ref, rout_ref, idx_ref):
    xb = x_ref[...]
    logits = jax.lax.dot_general(
        wc_ref[...], xb, (((1,), (1,)), ((), ())),
        preferred_element_type=jnp.float32) + bc_ref[...]
    e = eps_ref.shape[0]
    r = logits[:e]
    nl = logits[e:]
    noisy = r + eps_ref[...] * jnp.logaddexp(nl, 0.0)

    sub = jax.lax.broadcasted_iota(jnp.int32, noisy.shape, 0)
    m0 = jnp.max(noisy, axis=0, keepdims=True)
    i0 = jnp.min(jnp.where(noisy == m0, sub, e), axis=0, keepdims=True)
    masked = jnp.where(sub == i0, -jnp.inf, noisy)
    m1 = jnp.max(masked, axis=0, keepdims=True)
    i1 = jnp.min(jnp.where(masked == m1, sub, e), axis=0, keepdims=True)

    d = jnp.exp(m1 - m0)
    p0 = 1.0 / (1.0 + d)
    p1 = d / (1.0 + d)
    rout_t = (jnp.where(sub == i0, p0, 0.0) + jnp.where(sub == i1, p1, 0.0))
    idx_t = jnp.concatenate([i0, i1], axis=0)
    rout_ref[...] = rout_t.T
    idx_ref[...] = idx_t.T


def kernel(x, W_route, b_route, W_noise, b_noise):
    n, dim = x.shape
    e = W_route.shape[0]
    eps_t = _eps_const_t(n, e)
    wc = jnp.concatenate([W_route, W_noise], axis=0)
    bc = jnp.concatenate([b_route, b_noise]).reshape(2 * e, 1)
    blk = 2048
    out = pl.pallas_call(
        _router_kernel,
        grid=(n // blk,),
        in_specs=[
            pl.BlockSpec((blk, dim), lambda i: (i, 0)),
            pl.BlockSpec((2 * e, dim), lambda i: (0, 0)),
            pl.BlockSpec((2 * e, 1), lambda i: (0, 0)),
            pl.BlockSpec((e, blk), lambda i: (0, i)),
        ],
        out_specs=(
            pl.BlockSpec((blk, e), lambda i: (i, 0)),
            pl.BlockSpec((blk, 2), lambda i: (i, 0)),
        ),
        out_shape=(
            jax.ShapeDtypeStruct((n, e), jnp.float32),
            jax.ShapeDtypeStruct((n, 2), jnp.int32),
        ),
    )(x, wc, bc, eps_t)
    return out


# speedup vs baseline: 1.7688x; 1.6972x over previous
import functools

import jax
import jax.numpy as jnp
from jax.experimental import pallas as pl


@functools.lru_cache(maxsize=None)
def _eps_const_t(n, e):
    return jax.random.normal(jax.random.key(42), (n, e), dtype=jnp.float32).T


def _router_kernel(x_ref, wc_ref, bc_ref, eps_ref, rout_ref, idx_ref):
    xb = x_ref[...]
    logits = jax.lax.dot_general(
        wc_ref[...], xb, (((1,), (1,)), ((), ())),
        preferred_element_type=jnp.float32) + bc_ref[...]
    e = eps_ref.shape[0]
    r = logits[:e]
    nl = logits[e:]
    noisy = r + eps_ref[...] * jnp.logaddexp(nl, 0.0)

    sub = jax.lax.broadcasted_iota(jnp.int32, noisy.shape, 0)
    m0 = jnp.max(noisy, axis=0, keepdims=True)
    i0 = jnp.min(jnp.where(noisy == m0, sub, e), axis=0, keepdims=True)
    masked = jnp.where(sub == i0, -jnp.inf, noisy)
    m1 = jnp.max(masked, axis=0, keepdims=True)
    i1 = jnp.min(jnp.where(masked == m1, sub, e), axis=0, keepdims=True)

    d = jnp.exp(m1 - m0)
    p0 = 1.0 / (1.0 + d)
    p1 = d / (1.0 + d)
    rout_t = (jnp.where(sub == i0, p0, 0.0) + jnp.where(sub == i1, p1, 0.0))
    idx_t = jnp.concatenate([i0, i1], axis=0)
    rout_ref[...] = rout_t
    idx_ref[...] = idx_t


def kernel(x, W_route, b_route, W_noise, b_noise):
    n, dim = x.shape
    e = W_route.shape[0]
    eps_t = _eps_const_t(n, e)
    wc = jnp.concatenate([W_route, W_noise], axis=0)
    bc = jnp.concatenate([b_route, b_noise]).reshape(2 * e, 1)
    blk = 4096
    rout_t, idx_t = pl.pallas_call(
        _router_kernel,
        grid=(n // blk,),
        in_specs=[
            pl.BlockSpec((blk, dim), lambda i: (i, 0)),
            pl.BlockSpec((2 * e, dim), lambda i: (0, 0)),
            pl.BlockSpec((2 * e, 1), lambda i: (0, 0)),
            pl.BlockSpec((e, blk), lambda i: (0, i)),
        ],
        out_specs=(
            pl.BlockSpec((e, blk), lambda i: (0, i)),
            pl.BlockSpec((2, blk), lambda i: (0, i)),
        ),
        out_shape=(
            jax.ShapeDtypeStruct((e, n), jnp.float32),
            jax.ShapeDtypeStruct((2, n), jnp.int32),
        ),
    )(x, wc, bc, eps_t)
    return (rout_t.T, idx_t.T)
